# baseline (speedup 1.0000x reference)
import jax
import jax.numpy as jnp
from jax import lax
from jax.experimental import pallas as pl
from jax.experimental.pallas import tpu as pltpu
from jax.experimental.pallas import tpu_sc as plsc

N = 10000
E = 160000
NB = 100
DIN = 155
DE = 9
D = 256
H = 512
OUT = 4
L = 3

NC = 2
NS = 16
LN = 16
NW = NC * NS

NP = 10240
TPR = NP // NW
AGG_ROWS = TPR + 8
DUMMY = TPR
EB = 32
SB = 2000
FB = 512
RS = 4096
CAP = E + FB
RPAD = 128

_mesh = plsc.VectorSubcoreMesh(core_axis_name="c", subcore_axis_name="s")


def _prepass_body(src_hbm, dst_hbm, srcL, eidL, ldstL, counts,
                  dstblk, srcblk, st_s, st_e, st_d, cntbuf):
    c = lax.axis_index("c")
    s = lax.axis_index("s")
    wid = c * NS + s
    lo = wid * TPR
    iot = lax.iota(jnp.int32, LN)
    zero16 = iot * 0
    dummy16 = zero16 + DUMMY

    for g in range(2):
        reg = g * NW + wid

        def flush_one(flushed):
            fo = pl.multiple_of(flushed % RS, FB)
            dst0 = pl.multiple_of(reg * CAP + flushed, 8)
            pltpu.sync_copy(st_s.at[pl.ds(fo, FB)], srcL.at[pl.ds(dst0, FB)])
            pltpu.sync_copy(st_e.at[pl.ds(fo, FB)], eidL.at[pl.ds(dst0, FB)])
            pltpu.sync_copy(st_d.at[pl.ds(fo, FB)], ldstL.at[pl.ds(dst0, FB)])
            return flushed + FB

        def scan_block(j, carry):
            curv, flushed = carry
            base = pl.multiple_of(g * E + j * SB, 8)
            pltpu.sync_copy(dst_hbm.at[pl.ds(base, SB)], dstblk)
            pltpu.sync_copy(src_hbm.at[pl.ds(base, SB)], srcblk)

            @plsc.parallel_loop(0, SB // LN, 1, unroll=2, carry=curv)
            def chunk(k, cv):
                d16 = dstblk[pl.ds(k * LN, LN)]
                lv = d16 - lo
                ok = (lv >= 0) & (lv < TPR)
                oki = jnp.where(ok, 1, 0)
                pos = (cv + plsc.cumsum(oki) - 1) % RS
                s16 = srcblk[pl.ds(k * LN, LN)]
                e16 = (base + k * LN) + iot
                plsc.store_scatter(st_s, [pos], s16, mask=ok)
                plsc.store_scatter(st_e, [pos], e16, mask=ok)
                plsc.store_scatter(st_d, [pos], lv, mask=ok)
                return cv + plsc.all_reduce_population_count(ok)

            curv = chunk
            cs = jnp.sum(jnp.where(iot == 0, curv, 0))
            flushed = lax.while_loop(lambda f: cs - f >= FB, flush_one, flushed)
            return curv, flushed

        curv, flushed = lax.fori_loop(0, E // SB, scan_block, (zero16, 0))

        rnd16 = ((curv + (FB - 1)) // FB) * FB
        for kk in range(FB // LN):
            i16 = curv + kk * LN + iot
            mf = i16 < rnd16
            plsc.store_scatter(st_s, [i16 % RS], zero16, mask=mf)
            plsc.store_scatter(st_e, [i16 % RS], zero16, mask=mf)
            plsc.store_scatter(st_d, [i16 % RS], dummy16, mask=mf)
        padded = jnp.sum(jnp.where(iot == 0, rnd16, 0))
        flushed = lax.while_loop(lambda f: f < padded, flush_one, flushed)
        nb = padded // EB
        cntbuf[pl.ds(0, LN)] = zero16 + nb
        pltpu.sync_copy(cntbuf.at[pl.ds(0, 8)], counts.at[pl.ds(reg * 8, 8)])


_prepass_call = pl.kernel(
    _prepass_body,
    out_type=[
        jax.ShapeDtypeStruct((2 * NW * CAP,), jnp.int32),
        jax.ShapeDtypeStruct((2 * NW * CAP,), jnp.int32),
        jax.ShapeDtypeStruct((2 * NW * CAP,), jnp.int32),
        jax.ShapeDtypeStruct((2 * NW * 8,), jnp.int32),
    ],
    mesh=_mesh,
    compiler_params=pltpu.CompilerParams(needs_layout_passes=False),
    scratch_types=[
        pltpu.VMEM((SB,), jnp.int32),
        pltpu.VMEM((SB,), jnp.int32),
        pltpu.VMEM((RS,), jnp.int32),
        pltpu.VMEM((RS,), jnp.int32),
        pltpu.VMEM((RS,), jnp.int32),
        pltpu.VMEM((LN,), jnp.int32),
    ],
)


def _edge_body(h_hbm, e_hbm, srcL, eidL, ldstL, counts, z_hbm, agg_hbm,
               sidx0, sidx1, sidx2, eidx0, eidx1, eidx2,
               ldst0, ldst1, ldst2,
               hrows0, hrows1, erows0, erows1, aggv, cntv,
               semi0, semi1, semi2, semg0, semg1):
    c = lax.axis_index("c")
    s = lax.axis_index("s")
    wid = c * NS + s
    iot = lax.iota(jnp.int32, LN)
    sidx = (sidx0, sidx1, sidx2)
    eidx = (eidx0, eidx1, eidx2)
    ldst = (ldst0, ldst1, ldst2)
    hrows = (hrows0, hrows1)
    erows = (erows0, erows1)
    semi = (semi0, semi1, semi2)
    semg = (semg0, semg1)

    def graph_pass(g, carry0):
        reg = g * NW + wid
        pltpu.sync_copy(z_hbm, aggv)
        pltpu.sync_copy(counts.at[pl.ds(reg * 8, 8)], cntv.at[pl.ds(0, 8)])
        nb = jnp.sum(jnp.where(iot == 0, cntv[pl.ds(0, LN)], 0))
        base0 = reg * CAP

        def issue_idx(t, i3):
            b0 = pl.multiple_of(base0 + t * EB, 8)
            pltpu.async_copy(srcL.at[pl.ds(b0, EB)], sidx[i3], semi[i3])
            pltpu.async_copy(eidL.at[pl.ds(b0, EB)], eidx[i3], semi[i3])
            pltpu.async_copy(ldstL.at[pl.ds(b0, EB)],
                             ldst[i3].at[pl.ds(0, EB)], semi[i3])

        def wait_idx(t, i3):
            b0 = pl.multiple_of(base0 + t * EB, 8)
            pltpu.make_async_copy(srcL.at[pl.ds(b0, EB)], sidx[i3],
                                  semi[i3]).wait()
            pltpu.make_async_copy(eidL.at[pl.ds(b0, EB)], eidx[i3],
                                  semi[i3]).wait()
            pltpu.make_async_copy(ldstL.at[pl.ds(b0, EB)],
                                  ldst[i3].at[pl.ds(0, EB)], semi[i3]).wait()

        def issue_gather(i3, g2):
            pltpu.async_copy(h_hbm.at[sidx[i3]], hrows[g2], semg[g2])
            pltpu.async_copy(e_hbm.at[eidx[i3]], erows[g2], semg[g2])

        def wait_gather(i3, g2):
            pltpu.make_async_copy(h_hbm.at[sidx[i3]], hrows[g2],
                                  semg[g2]).wait()
            pltpu.make_async_copy(e_hbm.at[eidx[i3]], erows[g2],
                                  semg[g2]).wait()

        def compute(i3, g2):
            @plsc.parallel_loop(0, EB, 1, unroll=4)
            def row(r):
                dl = ldst[i3][pl.ds(r, LN)][0]
                for q in range(D // LN):
                    v = (hrows[g2][r, pl.ds(q * LN, LN)]
                         + erows[g2][r, pl.ds(q * LN, LN)])
                    plsc.addupdate(aggv.at[dl, pl.ds(q * LN, LN)],
                                   jnp.maximum(v, 0.0))

        @pl.when(nb > 0)
        def _():
            issue_idx(0, 0)
            wait_idx(0, 0)
            issue_gather(0, 0)

        @pl.when(nb > 1)
        def _():
            issue_idx(1, 1)

        @pl.when(nb > 2)
        def _():
            issue_idx(2, 2)

        def sixpack(ip, carry):
            for p in range(6):
                t = ip * 6 + p
                i3, g2 = p % 3, p % 2
                n3, ng = (p + 1) % 3, (p + 1) % 2

                @pl.when(t + 1 < nb)
                def _():
                    wait_idx(t + 1, n3)
                    issue_gather(n3, ng)

                @pl.when(t < nb)
                def _():
                    wait_gather(i3, g2)
                    compute(i3, g2)

                @pl.when(t + 3 < nb)
                def _():
                    issue_idx(t + 3, i3)

            return carry

        lax.fori_loop(0, (nb + 5) // 6, sixpack, 0)
        pltpu.sync_copy(aggv.at[pl.ds(0, TPR)],
                        agg_hbm.at[pl.ds(g * NP + wid * TPR, TPR)])
        return carry0

    lax.fori_loop(0, 2, graph_pass, 0)


_edge_call = pl.kernel(
    _edge_body,
    out_type=jax.ShapeDtypeStruct((2 * NP, D), jnp.float32),
    mesh=_mesh,
    compiler_params=pltpu.CompilerParams(needs_layout_passes=False),
    scratch_types=[
        pltpu.VMEM((EB,), jnp.int32),
        pltpu.VMEM((EB,), jnp.int32),
        pltpu.VMEM((EB,), jnp.int32),
        pltpu.VMEM((EB,), jnp.int32),
        pltpu.VMEM((EB,), jnp.int32),
        pltpu.VMEM((EB,), jnp.int32),
        pltpu.VMEM((EB + LN,), jnp.int32),
        pltpu.VMEM((EB + LN,), jnp.int32),
        pltpu.VMEM((EB + LN,), jnp.int32),
        pltpu.VMEM((EB, D), jnp.float32),
        pltpu.VMEM((EB, D), jnp.float32),
        pltpu.VMEM((EB, D), jnp.float32),
        pltpu.VMEM((EB, D), jnp.float32),
        pltpu.VMEM((AGG_ROWS, D), jnp.float32),
        pltpu.VMEM((LN,), jnp.int32),
        pltpu.SemaphoreType.DMA,
        pltpu.SemaphoreType.DMA,
        pltpu.SemaphoreType.DMA,
        pltpu.SemaphoreType.DMA,
        pltpu.SemaphoreType.DMA,
    ],
)


def _embed_nodes_body(x_ref, w_ref, b_ref, o_ref):
    o_ref[...] = jnp.dot(x_ref[...], w_ref[...],
                         preferred_element_type=jnp.float32) + b_ref[...]


def _embed_edges_body(a_ref, w_ref, b_ref, o_ref):
    v = jnp.dot(a_ref[...], w_ref[...],
                preferred_element_type=jnp.float32) + b_ref[...]
    o_ref[...] = jnp.maximum(v, 0.0)


def _layer_body(eps_ref, h_ref, agg_ref, acc_ref, w1_ref, b1_ref,
                w2_ref, b2_ref, h_out, acc_out):
    z = eps_ref[0, 0] * h_ref[...] + agg_ref[...]
    t = jnp.maximum(jnp.dot(z, w1_ref[...],
                            preferred_element_type=jnp.float32) + b1_ref[...], 0.0)
    hn = jnp.dot(t, w2_ref[...], preferred_element_type=jnp.float32) + b2_ref[...]
    h_out[...] = hn
    acc_out[...] = acc_ref[...] + hn


def _readout_body(b_ref, acc_ref, o_ref):
    i = pl.program_id(0)
    ids = lax.broadcasted_iota(jnp.int32, (1, 2 * RPAD), 1).astype(jnp.float32)
    onehot = jnp.where(b_ref[...] == ids, 1.0, 0.0)
    contrib = lax.dot_general(onehot, acc_ref[...],
                              dimension_numbers=(((0,), (0,)), ((), ())),
                              preferred_element_type=jnp.float32)

    @pl.when(i == 0)
    def _():
        o_ref[...] = jnp.zeros_like(o_ref)

    o_ref[...] += contrib


def _head_body(a1_ref, a2_ref, gf_ref, p1_ref, pb1_ref, p2_ref, pb2_ref,
               p3_ref, pb3_ref, o_ref):
    gf = gf_ref[...]
    f = gf[0:RPAD] - gf[RPAD:2 * RPAD]
    h1 = jnp.dot(f, p1_ref[...], preferred_element_type=jnp.float32) + pb1_ref[...]
    h1 = jnp.where(h1 >= 0, h1, a1_ref[0, 0] * h1)
    h2 = jnp.dot(h1, p2_ref[...], preferred_element_type=jnp.float32) + pb2_ref[...]
    h2 = jnp.where(h2 >= 0, h2, a2_ref[0, 0] * h2)
    o_ref[...] = jnp.dot(h2, p3_ref[...],
                         preferred_element_type=jnp.float32) + pb3_ref[...]


def _embed_nodes(x, w, b):
    return pl.pallas_call(
        _embed_nodes_body,
        grid=((2 * NP) // 1024,),
        in_specs=[
            pl.BlockSpec((1024, 160), lambda i: (i, 0)),
            pl.BlockSpec((160, D), lambda i: (0, 0)),
            pl.BlockSpec((1, D), lambda i: (0, 0)),
        ],
        out_specs=pl.BlockSpec((1024, D), lambda i: (i, 0)),
        out_shape=jax.ShapeDtypeStruct((2 * NP, D), jnp.float32),
    )(x, w, b)


def _embed_edges(a, w, b):
    return pl.pallas_call(
        _embed_edges_body,
        grid=((2 * E) // 800,),
        in_specs=[
            pl.BlockSpec((800, 16), lambda i: (i, 0)),
            pl.BlockSpec((16, D), lambda i: (0, 0)),
            pl.BlockSpec((1, D), lambda i: (0, 0)),
        ],
        out_specs=pl.BlockSpec((800, D), lambda i: (i, 0)),
        out_shape=jax.ShapeDtypeStruct((2 * E, D), jnp.float32),
    )(a, w, b)


def _layer_update(epsf, h, agg, acc, w1, b1, w2, b2):
    return pl.pallas_call(
        _layer_body,
        grid=((2 * NP) // 1024,),
        in_specs=[
            pl.BlockSpec(memory_space=pltpu.SMEM),
            pl.BlockSpec((1024, D), lambda i: (i, 0)),
            pl.BlockSpec((1024, D), lambda i: (i, 0)),
            pl.BlockSpec((1024, D), lambda i: (i, 0)),
            pl.BlockSpec((D, D), lambda i: (0, 0)),
            pl.BlockSpec((1, D), lambda i: (0, 0)),
            pl.BlockSpec((D, D), lambda i: (0, 0)),
            pl.BlockSpec((1, D), lambda i: (0, 0)),
        ],
        out_specs=[
            pl.BlockSpec((1024, D), lambda i: (i, 0)),
            pl.BlockSpec((1024, D), lambda i: (i, 0)),
        ],
        out_shape=[
            jax.ShapeDtypeStruct((2 * NP, D), jnp.float32),
            jax.ShapeDtypeStruct((2 * NP, D), jnp.float32),
        ],
    )(epsf, h, agg, acc, w1, b1, w2, b2)


def _readout(bf, acc):
    return pl.pallas_call(
        _readout_body,
        grid=((2 * NP) // 1024,),
        in_specs=[
            pl.BlockSpec((1024, 1), lambda i: (i, 0)),
            pl.BlockSpec((1024, D), lambda i: (i, 0)),
        ],
        out_specs=pl.BlockSpec((2 * RPAD, D), lambda i: (0, 0)),
        out_shape=jax.ShapeDtypeStruct((2 * RPAD, D), jnp.float32),
    )(bf, acc)


def _head(a1, a2, gf, p1, pb1, p2, pb2, p3p, pb3p):
    return pl.pallas_call(
        _head_body,
        grid=(1,),
        in_specs=[
            pl.BlockSpec(memory_space=pltpu.SMEM),
            pl.BlockSpec(memory_space=pltpu.SMEM),
            pl.BlockSpec((2 * RPAD, D), lambda i: (0, 0)),
            pl.BlockSpec((D, H), lambda i: (0, 0)),
            pl.BlockSpec((1, H), lambda i: (0, 0)),
            pl.BlockSpec((H, H), lambda i: (0, 0)),
            pl.BlockSpec((1, H), lambda i: (0, 0)),
            pl.BlockSpec((H, 128), lambda i: (0, 0)),
            pl.BlockSpec((1, 128), lambda i: (0, 0)),
        ],
        out_specs=pl.BlockSpec((RPAD, 128), lambda i: (0, 0)),
        out_shape=jax.ShapeDtypeStruct((RPAD, 128), jnp.float32),
    )(a1, a2, gf, p1, pb1, p2, pb2, p3p, pb3p)


def kernel(r_x, r_edge_index, r_edge_attr, r_batch,
           p_x, p_edge_index, p_edge_attr, p_batch,
           Wn, bn, We, be, W1s, b1s, W2s, b2s, eps,
           P1, pb1, a1, P2, pb2, a2, P3, pb3):
    f32 = jnp.float32

    x = jnp.zeros((2 * NP, 160), f32)
    x = x.at[0:N, 0:DIN].set(r_x).at[NP:NP + N, 0:DIN].set(p_x)
    wn_p = jnp.zeros((160, D), f32).at[0:DIN].set(Wn)

    ea = jnp.zeros((2 * E, 16), f32)
    ea = ea.at[0:E, 0:DE].set(r_edge_attr).at[E:2 * E, 0:DE].set(p_edge_attr)
    we_p = jnp.zeros((16, D), f32).at[0:DE].set(We)

    src = jnp.concatenate([r_edge_index[0], p_edge_index[0] + NP])
    dst = jnp.concatenate([r_edge_index[1], p_edge_index[1]])

    pad_b = jnp.full((NP - N,), NB, jnp.int32)
    bidx = jnp.concatenate([r_batch, pad_b, p_batch + RPAD, pad_b + RPAD])
    bf = bidx.astype(f32).reshape(2 * NP, 1)

    zeros_hbm = jnp.zeros((AGG_ROWS, D), f32)

    p3p = jnp.zeros((H, 128), f32).at[:, 0:OUT].set(P3)
    pb3p = jnp.zeros((1, 128), f32).at[0, 0:OUT].set(pb3)

    bn2 = bn.reshape(1, D)
    be2 = be.reshape(1, D)
    a1s = a1.reshape(1, 1)
    a2s = a2.reshape(1, 1)

    srcL, eidL, ldstL, counts = _prepass_call(src, dst)

    h = _embed_nodes(x, wn_p, bn2)
    e = _embed_edges(ea, we_p, be2)

    acc = jnp.zeros((2 * NP, D), f32)
    for l in range(L):
        agg = _edge_call(h, e, srcL, eidL, ldstL, counts, zeros_hbm)
        epsf = (1.0 + eps[l]).reshape(1, 1)
        h, acc = _layer_update(epsf, h, agg, acc,
                               W1s[l], b1s[l].reshape(1, D),
                               W2s[l], b2s[l].reshape(1, D))

    gf = _readout(bf, acc)
    out = _head(a1s, a2s, gf, P1, pb1.reshape(1, H),
                P2, pb2.reshape(1, H), p3p, pb3p)
    return out[0:NB, 0:OUT]
